# baseline (speedup 1.0000x reference)
import functools

import jax
import jax.numpy as jnp
from jax import lax
from jax.experimental import pallas as pl
from jax.experimental.pallas import tpu as pltpu
from jax.experimental.pallas import tpu_sc as plsc

VOCAB = 100000
K = 4
DIM = 2
N = 16384
ROW_W = K * DIM
ROW_B = K
PLANES = 2 * ROW_W + 2 * ROW_B

LANES = 16
HALF = N // 2
NVEC = HALF // LANES
OUTR = NVEC

_mesh = plsc.VectorSubcoreMesh(core_axis_name="c", subcore_axis_name="s")


@functools.partial(
    pl.kernel,
    mesh=_mesh,
    out_type=jax.ShapeDtypeStruct((PLANES, 2, OUTR, LANES), jnp.float32),
    scratch_types=[
        pltpu.VMEM((VOCAB,), jnp.float32),
        pltpu.VMEM((HALF,), jnp.int32),
        pltpu.VMEM((OUTR, LANES), jnp.float32),
        pltpu.VMEM((OUTR, LANES), jnp.float32),
        pltpu.SemaphoreType.DMA,
        pltpu.SemaphoreType.DMA,
    ],
    compiler_params=pltpu.CompilerParams(
        use_tc_tiling_on_sc=False, needs_layout_passes=False),
)
def _fire_word_gather(ranks_hbm, tab_hbm, out_hbm,
                      plane_v, idx_v, out0_v, out1_v, psem, osem):
    wid = lax.axis_index("s") * 2 + lax.axis_index("c")

    def gather_half(out_v):
        @plsc.parallel_loop(0, NVEC, unroll=8)
        def body(g):
            iv = idx_v[pl.ds(g * LANES, LANES)]
            out_v.at[g][...] = plsc.load_gather(plane_v, [iv])

    @pl.when(wid < PLANES)
    def _():
        pcopy = pltpu.async_copy(tab_hbm.at[wid], plane_v, psem)
        pltpu.sync_copy(ranks_hbm.at[pl.ds(0, HALF)], idx_v)
        pcopy.wait()
        gather_half(out0_v)
        o0 = pltpu.async_copy(out0_v, out_hbm.at[wid, 0], osem)
        pltpu.sync_copy(ranks_hbm.at[pl.ds(HALF, HALF)], idx_v)
        gather_half(out1_v)
        o1 = pltpu.async_copy(out1_v, out_hbm.at[wid, 1], osem)
        o0.wait()
        o1.wait()


def kernel(ranks, func_w, func_b, meas_x, meas_m):
    tab = jnp.concatenate([
        func_w.transpose(1, 2, 0).reshape(ROW_W, VOCAB),
        meas_x.transpose(1, 2, 0).reshape(ROW_W, VOCAB),
        func_b.transpose(1, 0),
        meas_m.transpose(1, 0),
    ])
    idx = ranks.astype(jnp.int32)
    out = _fire_word_gather(idx, tab).reshape(PLANES, N)
    fw = out[0:ROW_W].reshape(K, DIM, N).transpose(2, 0, 1)
    mx = out[ROW_W:2 * ROW_W].reshape(K, DIM, N).transpose(2, 0, 1)
    fb = out[2 * ROW_W:2 * ROW_W + ROW_B].transpose(1, 0)
    mm = out[2 * ROW_W + ROW_B:].transpose(1, 0)
    return fw, fb, mx, mm


# speedup vs baseline: 1.3820x; 1.3820x over previous
import functools

import jax
import jax.numpy as jnp
from jax import lax
from jax.experimental import pallas as pl
from jax.experimental.pallas import tpu as pltpu
from jax.experimental.pallas import tpu_sc as plsc

VOCAB = 100000
K = 4
DIM = 2
N = 16384
ROW_W = K * DIM
ROW_B = K

LANES = 16
HALF = N // 2
NVEC = HALF // LANES
OUTR = NVEC

_mesh = plsc.VectorSubcoreMesh(core_axis_name="c", subcore_axis_name="s")


@functools.partial(
    pl.kernel,
    mesh=_mesh,
    out_type=(
        jax.ShapeDtypeStruct((ROW_W, 2, OUTR, LANES), jnp.float32),
        jax.ShapeDtypeStruct((ROW_B, 2, OUTR, LANES), jnp.float32),
        jax.ShapeDtypeStruct((ROW_W, 2, OUTR, LANES), jnp.float32),
        jax.ShapeDtypeStruct((ROW_B, 2, OUTR, LANES), jnp.float32),
    ),
    scratch_types=[
        pltpu.VMEM((VOCAB,), jnp.float32),
        pltpu.VMEM((HALF,), jnp.int32),
        pltpu.VMEM((OUTR, LANES), jnp.float32),
        pltpu.VMEM((OUTR, LANES), jnp.float32),
        pltpu.SemaphoreType.DMA,
        pltpu.SemaphoreType.DMA,
    ],
    compiler_params=pltpu.CompilerParams(
        use_tc_tiling_on_sc=False, needs_layout_passes=False),
)
def _fire_word_gather(ranks_hbm, fw_hbm, fb_hbm, mx_hbm, mm_hbm,
                      ofw_hbm, ofb_hbm, omx_hbm, omm_hbm,
                      plane_v, idx_v, out0_v, out1_v, psem, osem):
    wid = lax.axis_index("s") * 2 + lax.axis_index("c")

    def gather_half(out_v):
        @plsc.parallel_loop(0, NVEC, unroll=8)
        def body(g):
            iv = idx_v[pl.ds(g * LANES, LANES)]
            out_v.at[g][...] = plsc.load_gather(plane_v, [iv])

    def do_table(tab_hbm, out_hbm, base, nplanes):
        @pl.when((wid >= base) & (wid < base + nplanes))
        def _():
            c = wid - base
            pcopy = pltpu.async_copy(tab_hbm.at[c], plane_v, psem)
            pltpu.sync_copy(ranks_hbm.at[pl.ds(0, HALF)], idx_v)
            pcopy.wait()
            gather_half(out0_v)
            o0 = pltpu.async_copy(out0_v, out_hbm.at[c, 0], osem)
            pltpu.sync_copy(ranks_hbm.at[pl.ds(HALF, HALF)], idx_v)
            gather_half(out1_v)
            o1 = pltpu.async_copy(out1_v, out_hbm.at[c, 1], osem)
            o0.wait()
            o1.wait()

    do_table(fw_hbm, ofw_hbm, 0, ROW_W)
    do_table(mx_hbm, omx_hbm, ROW_W, ROW_W)
    do_table(fb_hbm, ofb_hbm, 2 * ROW_W, ROW_B)
    do_table(mm_hbm, omm_hbm, 2 * ROW_W + ROW_B, ROW_B)


def kernel(ranks, func_w, func_b, meas_x, meas_m):
    fw_t = func_w.transpose(1, 2, 0).reshape(ROW_W, VOCAB)
    mx_t = meas_x.transpose(1, 2, 0).reshape(ROW_W, VOCAB)
    fb_t = func_b.transpose(1, 0)
    mm_t = meas_m.transpose(1, 0)
    idx = ranks.astype(jnp.int32)
    fw, fb, mx, mm = _fire_word_gather(idx, fw_t, fb_t, mx_t, mm_t)
    fw = fw.reshape(K, DIM, N).transpose(2, 0, 1)
    mx = mx.reshape(K, DIM, N).transpose(2, 0, 1)
    fb = fb.reshape(K, N).transpose(1, 0)
    mm = mm.reshape(K, N).transpose(1, 0)
    return fw, fb, mx, mm


# speedup vs baseline: 1.3885x; 1.0047x over previous
import functools

import jax
import jax.numpy as jnp
from jax import lax
from jax.experimental import pallas as pl
from jax.experimental.pallas import tpu as pltpu
from jax.experimental.pallas import tpu_sc as plsc

VOCAB = 100000
K = 4
DIM = 2
N = 16384
ROW_W = K * DIM
ROW_B = K
PLANES = 2 * ROW_W + 2 * ROW_B

LANES = 16
HALF = N // 2
NVEC = HALF // LANES
OUTR = NVEC

_mesh = plsc.VectorSubcoreMesh(core_axis_name="c", subcore_axis_name="s")


@functools.partial(
    pl.kernel,
    mesh=_mesh,
    out_type=(
        jax.ShapeDtypeStruct((ROW_W, 2, OUTR, LANES), jnp.float32),
        jax.ShapeDtypeStruct((ROW_B, 2, OUTR, LANES), jnp.float32),
        jax.ShapeDtypeStruct((ROW_W, 2, OUTR, LANES), jnp.float32),
        jax.ShapeDtypeStruct((ROW_B, 2, OUTR, LANES), jnp.float32),
    ),
    scratch_types=[
        pltpu.VMEM((VOCAB,), jnp.float32),
        pltpu.VMEM((HALF,), jnp.int32),
        pltpu.VMEM((OUTR, LANES), jnp.float32),
        pltpu.VMEM((OUTR, LANES), jnp.float32),
        pltpu.SemaphoreType.DMA,
        pltpu.SemaphoreType.DMA,
    ],
    compiler_params=pltpu.CompilerParams(
        use_tc_tiling_on_sc=False, needs_layout_passes=False),
)
def _fire_word_gather(ranks_hbm, fw_hbm, fb_hbm, mx_hbm, mm_hbm,
                      ofw_hbm, ofb_hbm, omx_hbm, omm_hbm,
                      plane_v, idx_v, out0_v, out1_v, psem, osem):
    wid = lax.axis_index("s") * 2 + lax.axis_index("c")

    tables = (
        (fw_hbm, ofw_hbm, 0, ROW_W),
        (mx_hbm, omx_hbm, ROW_W, ROW_W),
        (fb_hbm, ofb_hbm, 2 * ROW_W, ROW_B),
        (mm_hbm, omm_hbm, 2 * ROW_W + ROW_B, ROW_B),
    )

    def gather_half(out_v):
        @plsc.parallel_loop(0, NVEC, unroll=8)
        def body(g):
            iv = idx_v[pl.ds(g * LANES, LANES)]
            out_v.at[g][...] = plsc.load_gather(plane_v, [iv])

    def for_each_table(issue):
        for tab_hbm, out_hbm, base, nplanes in tables:
            @pl.when((wid >= base) & (wid < base + nplanes))
            def _():
                issue(tab_hbm, out_hbm, wid - base)

    for_each_table(
        lambda tab, out, c: pltpu.async_copy(tab.at[c], plane_v, psem))

    @pl.when(wid < PLANES)
    def _():
        pltpu.sync_copy(ranks_hbm.at[pl.ds(0, HALF)], idx_v)
        pltpu.make_async_copy(fw_hbm.at[0], plane_v, psem).wait()
        gather_half(out0_v)

    for_each_table(
        lambda tab, out, c: pltpu.async_copy(out0_v, out.at[c, 0], osem))

    @pl.when(wid < PLANES)
    def _():
        pltpu.sync_copy(ranks_hbm.at[pl.ds(HALF, HALF)], idx_v)
        gather_half(out1_v)

    for_each_table(
        lambda tab, out, c: pltpu.async_copy(out1_v, out.at[c, 1], osem))

    @pl.when(wid < PLANES)
    def _():
        pltpu.make_async_copy(ofw_hbm.at[0, 0], out0_v, osem).wait()
        pltpu.make_async_copy(ofw_hbm.at[0, 1], out1_v, osem).wait()


def kernel(ranks, func_w, func_b, meas_x, meas_m):
    fw_t = func_w.transpose(1, 2, 0).reshape(ROW_W, VOCAB)
    mx_t = meas_x.transpose(1, 2, 0).reshape(ROW_W, VOCAB)
    fb_t = func_b.transpose(1, 0)
    mm_t = meas_m.transpose(1, 0)
    idx = ranks.astype(jnp.int32)
    fw, fb, mx, mm = _fire_word_gather(idx, fw_t, fb_t, mx_t, mm_t)
    fw = fw.reshape(K, DIM, N).transpose(2, 0, 1)
    mx = mx.reshape(K, DIM, N).transpose(2, 0, 1)
    fb = fb.reshape(K, N).transpose(1, 0)
    mm = mm.reshape(K, N).transpose(1, 0)
    return fw, fb, mx, mm
